# baseline (speedup 1.0000x reference)
import functools

import jax
import jax.numpy as jnp
from jax import lax
from jax.experimental import pallas as pl
from jax.experimental.pallas import tpu as pltpu
from jax.experimental.pallas import tpu_sc as plsc

EPS = 1e-6
B, N, CIN, COUT = 4, 3136, 384, 768
H = W = 56
HO = WO = 28
P = H * W
NS = HO * WO
TN = 784
CH = COUT // 2

NWK = 28
TPW = 112


def _loc_to_idx(loc_orig, h, w):
    loc = jnp.clip(loc_orig, -1.0, 1.0)
    xs = 0.5 * (loc[..., 0] + 1.0) * w - 0.5
    ys = 0.5 * (loc[..., 1] + 1.0) * h - 0.5
    xg = jnp.clip(jnp.round(xs).astype(jnp.int32), 0, w - 1)
    yg = jnp.clip(jnp.round(ys).astype(jnp.int32), 0, h - 1)
    return yg * w + xg


_SC_MESH = plsc.VectorSubcoreMesh(core_axis_name="c", subcore_axis_name="s")


def _sc_gather_body(tab_hbm, idx_hbm, out_hbm, rows_v, idx_v, sem, *, c):
    cc = lax.axis_index("c")
    s = lax.axis_index("s")
    w = s * 2 + cc

    @pl.when(w < NWK)
    def _():
        for b in range(B):
            pltpu.sync_copy(idx_hbm.at[b, w], idx_v)
            pltpu.async_copy(tab_hbm.at[b].at[idx_v], rows_v, sem).wait()
            pltpu.sync_copy(rows_v, out_hbm.at[b, pl.ds(w * TPW, TPW)])


def _sc_gather(tab, idx_w, c):
    f = pl.kernel(
        functools.partial(_sc_gather_body, c=c),
        out_type=jax.ShapeDtypeStruct((B, N, c), jnp.float32),
        mesh=_SC_MESH,
        scratch_types=[
            pltpu.VMEM((TPW, c), jnp.float32),
            pltpu.VMEM((TPW,), jnp.int32),
            pltpu.SemaphoreType.DMA,
        ],
    )
    return f(tab, idx_w)


def _scatter_rows_body(idx_ref, z_ref, val_ref, o_ref, *, tp):
    pt = pl.program_id(1)
    idx = idx_ref[0, 0, :]
    sel = (idx[:, None] ==
           (pt * tp + jax.lax.broadcasted_iota(jnp.int32, (N, tp), 1)))
    val = val_ref[0, 0, :]
    sv = sel.astype(jnp.bfloat16) * val.astype(jnp.bfloat16)[:, None]
    acc = jax.lax.dot_general(sv, z_ref[0], (((0,), (0,)), ((), ())),
                              preferred_element_type=jnp.float32)
    ones = jnp.ones((N, 128), jnp.bfloat16)
    tot = jax.lax.dot_general(sv, ones, (((0,), (0,)), ((), ())),
                              preferred_element_type=jnp.float32)[:, :1]
    o_ref[0] = acc / (tot + EPS)


def _scatter_rows(idx3, z, val3, nbins, c, tp):
    return pl.pallas_call(
        functools.partial(_scatter_rows_body, tp=tp),
        grid=(B, nbins // tp),
        in_specs=[
            pl.BlockSpec((1, 1, N), lambda b, t: (b, 0, 0)),
            pl.BlockSpec((1, N, c), lambda b, t: (b, 0, 0)),
            pl.BlockSpec((1, 1, N), lambda b, t: (b, 0, 0)),
        ],
        out_specs=pl.BlockSpec((1, tp, c), lambda b, t: (b, t, 0)),
        out_shape=jax.ShapeDtypeStruct((B, nbins, c), jnp.float32),
    )(idx3, z, val3)


def _conv_body(xm_ref, w_ref, b_ref, o_ref):
    xr = xm_ref[0].astype(jnp.bfloat16).reshape(HO, 2, WO, 2, CIN)
    out = jnp.broadcast_to(b_ref[0, :], (NS, CH))

    def shifted(py, px, sy, sx):
        a = xr[:, py, :, px, :]
        if sy:
            a = jnp.concatenate([jnp.zeros((1, WO, CIN), jnp.bfloat16),
                                 a[:-1]], axis=0)
        if sx:
            a = jnp.concatenate([jnp.zeros((HO, 1, CIN), jnp.bfloat16),
                                 a[:, :-1]], axis=1)
        return a.reshape(NS, CIN)

    for ky, (py, sy) in enumerate(((1, True), (0, False), (1, False))):
        for kx, (px, sx) in enumerate(((1, True), (0, False), (1, False))):
            a = shifted(py, px, sy, sx)
            out = out + jnp.dot(a, w_ref[ky * 3 + kx],
                                preferred_element_type=jnp.float32)
    o_ref[0] = out


def _conv(xmap, w_taps, conv_b):
    return pl.pallas_call(
        _conv_body,
        grid=(B, COUT // CH),
        in_specs=[
            pl.BlockSpec((1, P, CIN), lambda b, h: (b, 0, 0)),
            pl.BlockSpec((9, CIN, CH), lambda b, h: (0, 0, h)),
            pl.BlockSpec((1, CH), lambda b, h: (0, h)),
        ],
        out_specs=pl.BlockSpec((1, NS, CH), lambda b, h: (b, 0, h)),
        out_shape=jax.ShapeDtypeStruct((B, NS, COUT), jnp.float32),
    )(xmap, w_taps, conv_b.reshape(1, COUT))


def _xt_stats_body(m2t_ref, x_ref, sw_ref, xt_ref, s_ref):
    xt = (m2t_ref[0]
          + jnp.dot(x_ref[0].astype(jnp.bfloat16), sw_ref[...],
                    preferred_element_type=jnp.float32))
    xt_ref[0] = xt
    s0 = jnp.sum(xt, axis=0)
    s1 = jnp.sum(xt * xt, axis=0)
    st = jnp.stack([s0, s1], axis=0)
    first = (pl.program_id(0) == 0) & (pl.program_id(1) == 0)

    @pl.when(first)
    def _():
        s_ref[...] = st

    @pl.when(jnp.logical_not(first))
    def _():
        s_ref[...] += st


def _xt_stats(m2t, x, skip_wt):
    return pl.pallas_call(
        _xt_stats_body,
        grid=(B, N // TN),
        in_specs=[
            pl.BlockSpec((1, TN, COUT), lambda b, t: (b, t, 0)),
            pl.BlockSpec((1, TN, CIN), lambda b, t: (b, t, 0)),
            pl.BlockSpec((CIN, COUT), lambda b, t: (0, 0)),
        ],
        out_specs=[
            pl.BlockSpec((1, TN, COUT), lambda b, t: (b, t, 0)),
            pl.BlockSpec((2, COUT), lambda b, t: (0, 0)),
        ],
        out_shape=[
            jax.ShapeDtypeStruct((B, N, COUT), jnp.float32),
            jax.ShapeDtypeStruct((2, COUT), jnp.float32),
        ],
    )(m2t, x, skip_wt)


TA = N // 2
HA = HO // 2


def _apply_body(xt_ref, s_ref, g_ref, bt_ref, cw_ref, cb_ref,
                xtn_ref, conf_ref, xd_ref, nw_ref):
    cnt = B * N
    mean = s_ref[0, :] / cnt
    var = s_ref[1, :] / cnt - mean * mean
    inv = jax.lax.rsqrt(var + 1e-5)
    xtn = (xt_ref[0] - mean[None, :]) * (inv * g_ref[0, :])[None, :] \
        + bt_ref[0, :][None, :]
    conf = jnp.sum(xtn * cw_ref[0, :][None, :], axis=1) + cb_ref[0, 0]
    wgt = jnp.exp(conf)
    wg = wgt.reshape(HA, 2, WO, 2)
    aw = wg.sum(axis=(1, 3)) + EPS
    nw = wg / aw[:, None, :, None]
    xd = (xtn.reshape(HA, 2, WO, 2, COUT)
          * nw[..., None]).sum(axis=(1, 3)).reshape(HA * WO, COUT)
    xtn_ref[0] = jax.nn.relu(xtn)
    conf_ref[0] = conf[:, None]
    xd_ref[0] = jax.nn.relu(xd)
    nw_ref[0, 0, 0, :] = nw.reshape(TA)


def _apply(xt, stats, gamma, beta, conf_w, conf_b):
    return pl.pallas_call(
        _apply_body,
        grid=(B, N // TA),
        in_specs=[
            pl.BlockSpec((1, TA, COUT), lambda b, t: (b, t, 0)),
            pl.BlockSpec((2, COUT), lambda b, t: (0, 0)),
            pl.BlockSpec((1, COUT), lambda b, t: (0, 0)),
            pl.BlockSpec((1, COUT), lambda b, t: (0, 0)),
            pl.BlockSpec((1, COUT), lambda b, t: (0, 0)),
            pl.BlockSpec((1, 1), lambda b, t: (0, 0)),
        ],
        out_specs=[
            pl.BlockSpec((1, TA, COUT), lambda b, t: (b, t, 0)),
            pl.BlockSpec((1, TA, 1), lambda b, t: (b, t, 0)),
            pl.BlockSpec((1, HA * WO, COUT), lambda b, t: (b, t, 0)),
            pl.BlockSpec((1, 1, 1, TA), lambda b, t: (b, t, 0, 0)),
        ],
        out_shape=[
            jax.ShapeDtypeStruct((B, N, COUT), jnp.float32),
            jax.ShapeDtypeStruct((B, N, 1), jnp.float32),
            jax.ShapeDtypeStruct((B, NS, COUT), jnp.float32),
            jax.ShapeDtypeStruct((B, N // TA, 1, TA), jnp.float32),
        ],
    )(xt, stats, gamma.reshape(1, COUT), beta.reshape(1, COUT),
      conf_w.reshape(1, COUT), conf_b.reshape(1, 1))


def _final_body(idx_ref, nw_ref, aw_ref, awd_ref, iad_ref):
    idx = idx_ref[0, 0, :]
    nw = nw_ref[0, 0, :]
    wt = jnp.zeros((N,), jnp.float32)
    for t in range(N // TN):
        sel = (idx[:, None] ==
               (t * TN + jax.lax.broadcasted_iota(jnp.int32, (N, TN), 1)))
        part = jnp.sum(sel.astype(jnp.float32)
                       * nw[t * TN:(t + 1) * TN][None, :], axis=1)
        wt = wt + part
    awd = aw_ref[0, 0, :] * wt
    awd = awd / jnp.max(awd)
    awd_ref[0, 0, :] = awd
    iad_ref[0, 0, :] = ((idx // W) // 2) * WO + (idx % W) // 2


def _final(idx3, nw3, aw3):
    return pl.pallas_call(
        _final_body,
        grid=(B,),
        in_specs=[
            pl.BlockSpec((1, 1, N), lambda b: (b, 0, 0)),
            pl.BlockSpec((1, 1, N), lambda b: (b, 0, 0)),
            pl.BlockSpec((1, 1, N), lambda b: (b, 0, 0)),
        ],
        out_specs=[
            pl.BlockSpec((1, 1, N), lambda b: (b, 0, 0)),
            pl.BlockSpec((1, 1, N), lambda b: (b, 0, 0)),
        ],
        out_shape=[
            jax.ShapeDtypeStruct((B, 1, N), jnp.float32),
            jax.ShapeDtypeStruct((B, 1, N), jnp.int32),
        ],
    )(idx3, nw3, aw3)


def kernel(x, loc_orig, idx_agg, agg_weight, map_h, map_w,
           conv_w, conv_b, skip_w, gamma, beta, conf_w, conf_b):
    idx_agg = idx_agg.astype(jnp.int32)
    idx56 = _loc_to_idx(loc_orig, H, W)
    idx28 = _loc_to_idx(loc_orig, HO, WO)
    ia3 = idx_agg.reshape(B, 1, N)
    ones3 = jnp.ones((B, 1, N), jnp.float32)
    val3 = agg_weight.reshape(B, 1, N)
    iag_w = idx_agg.reshape(B, NWK, TPW)
    i28_w = idx28.reshape(B, NWK, TPW)

    feat = _sc_gather(x, iag_w, CIN)
    xmap = _scatter_rows(idx56.reshape(B, 1, N), feat.astype(jnp.bfloat16),
                         ones3, P, CIN, TN)

    w_taps = conv_w.transpose(2, 3, 1, 0).reshape(9, CIN, COUT)
    w_taps = w_taps.astype(jnp.bfloat16)
    ymap = _conv(xmap, w_taps, conv_b)

    feat28 = _sc_gather(ymap, i28_w, COUT)
    m2t = _scatter_rows(ia3, feat28.astype(jnp.bfloat16), val3, N, COUT, TN)

    xt, stats = _xt_stats(m2t, x, skip_w.T.astype(jnp.bfloat16))

    xtn, conf, xd, nw4 = _apply(xt, stats, gamma, beta, conf_w, conf_b)

    awd3, iad3 = _final(ia3, nw4.reshape(B, 1, N), val3)

    return (xd, iad3.reshape(B, N), awd3.reshape(B, N, 1),
            conf, xtn)


# speedup vs baseline: 1.0217x; 1.0217x over previous
import functools

import jax
import jax.numpy as jnp
from jax import lax
from jax.experimental import pallas as pl
from jax.experimental.pallas import tpu as pltpu
from jax.experimental.pallas import tpu_sc as plsc

EPS = 1e-6
B, N, CIN, COUT = 4, 3136, 384, 768
H = W = 56
HO = WO = 28
P = H * W
NS = HO * WO
TN = 784
CH = COUT // 2

NWK = 28
TPW = 112


def _loc_to_idx(loc_orig, h, w):
    loc = jnp.clip(loc_orig, -1.0, 1.0)
    xs = 0.5 * (loc[..., 0] + 1.0) * w - 0.5
    ys = 0.5 * (loc[..., 1] + 1.0) * h - 0.5
    xg = jnp.clip(jnp.round(xs).astype(jnp.int32), 0, w - 1)
    yg = jnp.clip(jnp.round(ys).astype(jnp.int32), 0, h - 1)
    return yg * w + xg


_SC_MESH = plsc.VectorSubcoreMesh(core_axis_name="c", subcore_axis_name="s")


def _sc_gather_body(tab_hbm, idx_hbm, out_hbm, rows_v, idx_v, sem, *, c):
    cc = lax.axis_index("c")
    s = lax.axis_index("s")
    w = s * 2 + cc

    @pl.when(w < NWK)
    def _():
        for b in range(B):
            pltpu.sync_copy(idx_hbm.at[b, w], idx_v)
            pltpu.async_copy(tab_hbm.at[b].at[idx_v], rows_v, sem).wait()
            pltpu.sync_copy(rows_v, out_hbm.at[b, pl.ds(w * TPW, TPW)])


def _sc_gather(tab, idx_w, c):
    f = pl.kernel(
        functools.partial(_sc_gather_body, c=c),
        out_type=jax.ShapeDtypeStruct((B, N, c), jnp.float32),
        mesh=_SC_MESH,
        scratch_types=[
            pltpu.VMEM((TPW, c), jnp.float32),
            pltpu.VMEM((TPW,), jnp.int32),
            pltpu.SemaphoreType.DMA,
        ],
    )
    return f(tab, idx_w)


def _scatter_rows_body(idx_ref, z_ref, val_ref, o_ref, *, tp):
    pt = pl.program_id(1)
    idx = idx_ref[0, 0, :]
    sel = (idx[:, None] ==
           (pt * tp + jax.lax.broadcasted_iota(jnp.int32, (N, tp), 1)))
    val = val_ref[0, 0, :]
    sv = sel.astype(jnp.bfloat16) * val.astype(jnp.bfloat16)[:, None]
    acc = jax.lax.dot_general(sv, z_ref[0], (((0,), (0,)), ((), ())),
                              preferred_element_type=jnp.float32)
    ones = jnp.ones((N, 128), jnp.bfloat16)
    tot = jax.lax.dot_general(sv, ones, (((0,), (0,)), ((), ())),
                              preferred_element_type=jnp.float32)[:, :1]
    o_ref[0] = acc / (tot + EPS)


def _scatter_rows(idx3, z, val3, nbins, c, tp):
    return pl.pallas_call(
        functools.partial(_scatter_rows_body, tp=tp),
        grid=(B, nbins // tp),
        in_specs=[
            pl.BlockSpec((1, 1, N), lambda b, t: (b, 0, 0)),
            pl.BlockSpec((1, N, c), lambda b, t: (b, 0, 0)),
            pl.BlockSpec((1, 1, N), lambda b, t: (b, 0, 0)),
        ],
        out_specs=pl.BlockSpec((1, tp, c), lambda b, t: (b, t, 0)),
        out_shape=jax.ShapeDtypeStruct((B, nbins, c), jnp.float32),
    )(idx3, z, val3)


def _conv_body(xm_ref, w_ref, b_ref, o_ref):
    xr = xm_ref[0].reshape(HO, 2, WO, 2, CIN)
    out = jnp.broadcast_to(b_ref[0, :], (NS, CH))

    def shifted(py, px, sy, sx):
        a = xr[:, py, :, px, :]
        if sy:
            a = jnp.concatenate([jnp.zeros((1, WO, CIN), jnp.float32),
                                 a[:-1]], axis=0)
        if sx:
            a = jnp.concatenate([jnp.zeros((HO, 1, CIN), jnp.float32),
                                 a[:, :-1]], axis=1)
        return a.reshape(NS, CIN)

    for ky, (py, sy) in enumerate(((1, True), (0, False), (1, False))):
        for kx, (px, sx) in enumerate(((1, True), (0, False), (1, False))):
            a = shifted(py, px, sy, sx)
            out = out + jnp.dot(a, w_ref[ky * 3 + kx],
                                preferred_element_type=jnp.float32)
    o_ref[0] = out


def _conv(xmap, w_taps, conv_b):
    return pl.pallas_call(
        _conv_body,
        grid=(B, COUT // CH),
        in_specs=[
            pl.BlockSpec((1, P, CIN), lambda b, h: (b, 0, 0)),
            pl.BlockSpec((9, CIN, CH), lambda b, h: (0, 0, h)),
            pl.BlockSpec((1, CH), lambda b, h: (0, h)),
        ],
        out_specs=pl.BlockSpec((1, NS, CH), lambda b, h: (b, 0, h)),
        out_shape=jax.ShapeDtypeStruct((B, NS, COUT), jnp.float32),
    )(xmap, w_taps, conv_b.reshape(1, COUT))


def _xt_stats_body(m2t_ref, x_ref, sw_ref, xt_ref, s_ref):
    xt = (m2t_ref[0]
          + jnp.dot(x_ref[0], sw_ref[...],
                    preferred_element_type=jnp.float32))
    xt_ref[0] = xt
    s0 = jnp.sum(xt, axis=0)
    s1 = jnp.sum(xt * xt, axis=0)
    st = jnp.stack([s0, s1], axis=0)
    first = (pl.program_id(0) == 0) & (pl.program_id(1) == 0)

    @pl.when(first)
    def _():
        s_ref[...] = st

    @pl.when(jnp.logical_not(first))
    def _():
        s_ref[...] += st


def _xt_stats(m2t, x, skip_wt):
    return pl.pallas_call(
        _xt_stats_body,
        grid=(B, N // TN),
        in_specs=[
            pl.BlockSpec((1, TN, COUT), lambda b, t: (b, t, 0)),
            pl.BlockSpec((1, TN, CIN), lambda b, t: (b, t, 0)),
            pl.BlockSpec((CIN, COUT), lambda b, t: (0, 0)),
        ],
        out_specs=[
            pl.BlockSpec((1, TN, COUT), lambda b, t: (b, t, 0)),
            pl.BlockSpec((2, COUT), lambda b, t: (0, 0)),
        ],
        out_shape=[
            jax.ShapeDtypeStruct((B, N, COUT), jnp.float32),
            jax.ShapeDtypeStruct((2, COUT), jnp.float32),
        ],
    )(m2t, x, skip_wt)


TA = N // 2
HA = HO // 2


def _apply_body(xt_ref, s_ref, g_ref, bt_ref, cw_ref, cb_ref,
                xtn_ref, conf_ref, xd_ref, nw_ref):
    cnt = B * N
    mean = s_ref[0, :] / cnt
    var = s_ref[1, :] / cnt - mean * mean
    inv = jax.lax.rsqrt(var + 1e-5)
    xtn = (xt_ref[0] - mean[None, :]) * (inv * g_ref[0, :])[None, :] \
        + bt_ref[0, :][None, :]
    conf = jnp.sum(xtn * cw_ref[0, :][None, :], axis=1) + cb_ref[0, 0]
    wgt = jnp.exp(conf)
    wg = wgt.reshape(HA, 2, WO, 2)
    aw = wg.sum(axis=(1, 3)) + EPS
    nw = wg / aw[:, None, :, None]
    xd = (xtn.reshape(HA, 2, WO, 2, COUT)
          * nw[..., None]).sum(axis=(1, 3)).reshape(HA * WO, COUT)
    xtn_ref[0] = jax.nn.relu(xtn)
    conf_ref[0] = conf[:, None]
    xd_ref[0] = jax.nn.relu(xd)
    nw_ref[0, 0, 0, :] = nw.reshape(TA)


def _apply(xt, stats, gamma, beta, conf_w, conf_b):
    return pl.pallas_call(
        _apply_body,
        grid=(B, N // TA),
        in_specs=[
            pl.BlockSpec((1, TA, COUT), lambda b, t: (b, t, 0)),
            pl.BlockSpec((2, COUT), lambda b, t: (0, 0)),
            pl.BlockSpec((1, COUT), lambda b, t: (0, 0)),
            pl.BlockSpec((1, COUT), lambda b, t: (0, 0)),
            pl.BlockSpec((1, COUT), lambda b, t: (0, 0)),
            pl.BlockSpec((1, 1), lambda b, t: (0, 0)),
        ],
        out_specs=[
            pl.BlockSpec((1, TA, COUT), lambda b, t: (b, t, 0)),
            pl.BlockSpec((1, TA, 1), lambda b, t: (b, t, 0)),
            pl.BlockSpec((1, HA * WO, COUT), lambda b, t: (b, t, 0)),
            pl.BlockSpec((1, 1, 1, TA), lambda b, t: (b, t, 0, 0)),
        ],
        out_shape=[
            jax.ShapeDtypeStruct((B, N, COUT), jnp.float32),
            jax.ShapeDtypeStruct((B, N, 1), jnp.float32),
            jax.ShapeDtypeStruct((B, NS, COUT), jnp.float32),
            jax.ShapeDtypeStruct((B, N // TA, 1, TA), jnp.float32),
        ],
    )(xt, stats, gamma.reshape(1, COUT), beta.reshape(1, COUT),
      conf_w.reshape(1, COUT), conf_b.reshape(1, 1))


def _final_body(idx_ref, nw_ref, aw_ref, awd_ref, iad_ref):
    idx = idx_ref[0, 0, :]
    nw = nw_ref[0, 0, :]
    wt = jnp.zeros((N,), jnp.float32)
    for t in range(N // TN):
        sel = (idx[:, None] ==
               (t * TN + jax.lax.broadcasted_iota(jnp.int32, (N, TN), 1)))
        part = jnp.sum(sel.astype(jnp.float32)
                       * nw[t * TN:(t + 1) * TN][None, :], axis=1)
        wt = wt + part
    awd = aw_ref[0, 0, :] * wt
    awd = awd / jnp.max(awd)
    awd_ref[0, 0, :] = awd
    iad_ref[0, 0, :] = ((idx // W) // 2) * WO + (idx % W) // 2


def _final(idx3, nw3, aw3):
    return pl.pallas_call(
        _final_body,
        grid=(B,),
        in_specs=[
            pl.BlockSpec((1, 1, N), lambda b: (b, 0, 0)),
            pl.BlockSpec((1, 1, N), lambda b: (b, 0, 0)),
            pl.BlockSpec((1, 1, N), lambda b: (b, 0, 0)),
        ],
        out_specs=[
            pl.BlockSpec((1, 1, N), lambda b: (b, 0, 0)),
            pl.BlockSpec((1, 1, N), lambda b: (b, 0, 0)),
        ],
        out_shape=[
            jax.ShapeDtypeStruct((B, 1, N), jnp.float32),
            jax.ShapeDtypeStruct((B, 1, N), jnp.int32),
        ],
    )(idx3, nw3, aw3)


def kernel(x, loc_orig, idx_agg, agg_weight, map_h, map_w,
           conv_w, conv_b, skip_w, gamma, beta, conf_w, conf_b):
    idx_agg = idx_agg.astype(jnp.int32)
    idx56 = _loc_to_idx(loc_orig, H, W)
    idx28 = _loc_to_idx(loc_orig, HO, WO)
    ia3 = idx_agg.reshape(B, 1, N)
    ones3 = jnp.ones((B, 1, N), jnp.float32)
    val3 = agg_weight.reshape(B, 1, N)
    iag_w = idx_agg.reshape(B, NWK, TPW)
    i28_w = idx28.reshape(B, NWK, TPW)

    feat = _sc_gather(x, iag_w, CIN)
    xmap = _scatter_rows(idx56.reshape(B, 1, N), feat.astype(jnp.bfloat16),
                         ones3, P, CIN, TN)

    w_taps = conv_w.transpose(2, 3, 1, 0).reshape(9, CIN, COUT)
    ymap = _conv(xmap, w_taps, conv_b)

    feat28 = _sc_gather(ymap, i28_w, COUT)
    m2t = _scatter_rows(ia3, feat28.astype(jnp.bfloat16), val3, N, COUT, TN)

    xt, stats = _xt_stats(m2t, x, skip_w.T)

    xtn, conf, xd, nw4 = _apply(xt, stats, gamma, beta, conf_w, conf_b)

    awd3, iad3 = _final(ia3, nw4.reshape(B, 1, N), val3)

    return (xd, iad3.reshape(B, N), awd3.reshape(B, N, 1),
            conf, xtn)


# speedup vs baseline: 1.1526x; 1.1281x over previous
import functools

import jax
import jax.numpy as jnp
from jax import lax
from jax.experimental import pallas as pl
from jax.experimental.pallas import tpu as pltpu
from jax.experimental.pallas import tpu_sc as plsc

EPS = 1e-6
B, N, CIN, COUT = 4, 3136, 384, 768
H = W = 56
HO = WO = 28
P = H * W
NS = HO * WO
TN = 784
CH = COUT // 2

NWK = 28
TPW = 112


def _loc_to_idx(loc_orig, h, w):
    loc = jnp.clip(loc_orig, -1.0, 1.0)
    xs = 0.5 * (loc[..., 0] + 1.0) * w - 0.5
    ys = 0.5 * (loc[..., 1] + 1.0) * h - 0.5
    xg = jnp.clip(jnp.round(xs).astype(jnp.int32), 0, w - 1)
    yg = jnp.clip(jnp.round(ys).astype(jnp.int32), 0, h - 1)
    return yg * w + xg


_SC_MESH = plsc.VectorSubcoreMesh(core_axis_name="c", subcore_axis_name="s")


def _sc_gather_body(tab_hbm, idx_hbm, out_hbm, rows_v, idx_v, sem, *, c):
    cc = lax.axis_index("c")
    s = lax.axis_index("s")
    w = s * 2 + cc

    @pl.when(w < NWK)
    def _():
        for b in range(B):
            pltpu.sync_copy(idx_hbm.at[b, w], idx_v)
            pltpu.async_copy(tab_hbm.at[b].at[idx_v], rows_v, sem).wait()
            pltpu.sync_copy(rows_v, out_hbm.at[b, pl.ds(w * TPW, TPW)])


def _sc_gather(tab, idx_w, c):
    f = pl.kernel(
        functools.partial(_sc_gather_body, c=c),
        out_type=jax.ShapeDtypeStruct((B, N, c), jnp.float32),
        mesh=_SC_MESH,
        scratch_types=[
            pltpu.VMEM((TPW, c), jnp.float32),
            pltpu.VMEM((TPW,), jnp.int32),
            pltpu.SemaphoreType.DMA,
        ],
    )
    return f(tab, idx_w)


def _scatter_rows_body(idx_ref, z_ref, val_ref, o_ref, *, tp):
    pt = pl.program_id(1)
    idx = idx_ref[0, 0, :]
    sel = (idx[:, None] ==
           (pt * tp + jax.lax.broadcasted_iota(jnp.int32, (N, tp), 1)))
    val = val_ref[0, 0, :]
    sv = sel.astype(jnp.bfloat16) * val.astype(jnp.bfloat16)[:, None]
    acc = jax.lax.dot_general(sv, z_ref[0].astype(jnp.bfloat16),
                              (((0,), (0,)), ((), ())),
                              preferred_element_type=jnp.float32)
    ones = jnp.ones((N, 128), jnp.bfloat16)
    tot = jax.lax.dot_general(sv, ones, (((0,), (0,)), ((), ())),
                              preferred_element_type=jnp.float32)[:, :1]
    o_ref[0] = acc / (tot + EPS)


def _scatter_rows(idx3, z, val3, nbins, c, tp):
    return pl.pallas_call(
        functools.partial(_scatter_rows_body, tp=tp),
        grid=(B, nbins // tp),
        in_specs=[
            pl.BlockSpec((1, 1, N), lambda b, t: (b, 0, 0)),
            pl.BlockSpec((1, N, c), lambda b, t: (b, 0, 0)),
            pl.BlockSpec((1, 1, N), lambda b, t: (b, 0, 0)),
        ],
        out_specs=pl.BlockSpec((1, tp, c), lambda b, t: (b, t, 0)),
        out_shape=jax.ShapeDtypeStruct((B, nbins, c), jnp.float32),
    )(idx3, z, val3)


def _conv_body(xm_ref, w_ref, b_ref, o_ref):
    xr = xm_ref[0].reshape(HO, 2, WO, 2, CIN)
    out = jnp.broadcast_to(b_ref[0, :], (NS, CH))

    def shifted(py, px, sy, sx):
        a = xr[:, py, :, px, :]
        if sy:
            a = jnp.concatenate([jnp.zeros((1, WO, CIN), jnp.float32),
                                 a[:-1]], axis=0)
        if sx:
            a = jnp.concatenate([jnp.zeros((HO, 1, CIN), jnp.float32),
                                 a[:, :-1]], axis=1)
        return a.reshape(NS, CIN)

    for ky, (py, sy) in enumerate(((1, True), (0, False), (1, False))):
        for kx, (px, sx) in enumerate(((1, True), (0, False), (1, False))):
            a = shifted(py, px, sy, sx)
            out = out + jnp.dot(a, w_ref[ky * 3 + kx],
                                preferred_element_type=jnp.float32)
    o_ref[0] = out


def _conv(xmap, w_taps, conv_b):
    return pl.pallas_call(
        _conv_body,
        grid=(B, COUT // CH),
        in_specs=[
            pl.BlockSpec((1, P, CIN), lambda b, h: (b, 0, 0)),
            pl.BlockSpec((9, CIN, CH), lambda b, h: (0, 0, h)),
            pl.BlockSpec((1, CH), lambda b, h: (0, h)),
        ],
        out_specs=pl.BlockSpec((1, NS, CH), lambda b, h: (b, 0, h)),
        out_shape=jax.ShapeDtypeStruct((B, NS, COUT), jnp.float32),
    )(xmap, w_taps, conv_b.reshape(1, COUT))


def _m2t_xt_body(idx_ref, z_ref, val_ref, x_ref, sw_ref, xt_ref, s_ref):
    pt = pl.program_id(1)
    idx = idx_ref[0, 0, :]
    sel = (idx[:, None] ==
           (pt * TN + jax.lax.broadcasted_iota(jnp.int32, (N, TN), 1)))
    val = val_ref[0, 0, :]
    sv = sel.astype(jnp.bfloat16) * val.astype(jnp.bfloat16)[:, None]
    acc = jax.lax.dot_general(sv, z_ref[0].astype(jnp.bfloat16),
                              (((0,), (0,)), ((), ())),
                              preferred_element_type=jnp.float32)
    ones = jnp.ones((N, 128), jnp.bfloat16)
    tot = jax.lax.dot_general(sv, ones, (((0,), (0,)), ((), ())),
                              preferred_element_type=jnp.float32)[:, :1]
    xt = (acc / (tot + EPS)
          + jnp.dot(x_ref[0], sw_ref[...],
                    preferred_element_type=jnp.float32))
    xt_ref[0] = xt
    s0 = jnp.sum(xt, axis=0)
    s1 = jnp.sum(xt * xt, axis=0)
    st = jnp.stack([s0, s1], axis=0)
    first = (pl.program_id(0) == 0) & (pl.program_id(1) == 0)

    @pl.when(first)
    def _():
        s_ref[...] = st

    @pl.when(jnp.logical_not(first))
    def _():
        s_ref[...] += st


def _m2t_xt(idx3, z, val3, x, skip_wt):
    return pl.pallas_call(
        _m2t_xt_body,
        grid=(B, N // TN),
        in_specs=[
            pl.BlockSpec((1, 1, N), lambda b, t: (b, 0, 0)),
            pl.BlockSpec((1, N, COUT), lambda b, t: (b, 0, 0)),
            pl.BlockSpec((1, 1, N), lambda b, t: (b, 0, 0)),
            pl.BlockSpec((1, TN, CIN), lambda b, t: (b, t, 0)),
            pl.BlockSpec((CIN, COUT), lambda b, t: (0, 0)),
        ],
        out_specs=[
            pl.BlockSpec((1, TN, COUT), lambda b, t: (b, t, 0)),
            pl.BlockSpec((2, COUT), lambda b, t: (0, 0)),
        ],
        out_shape=[
            jax.ShapeDtypeStruct((B, N, COUT), jnp.float32),
            jax.ShapeDtypeStruct((2, COUT), jnp.float32),
        ],
    )(idx3, z, val3, x, skip_wt)


TA = N // 2
HA = HO // 2


def _apply_body(xt_ref, s_ref, g_ref, bt_ref, cw_ref, cb_ref,
                xtn_ref, conf_ref, xd_ref, nw_ref):
    cnt = B * N
    mean = s_ref[0, :] / cnt
    var = s_ref[1, :] / cnt - mean * mean
    inv = jax.lax.rsqrt(var + 1e-5)
    xtn = (xt_ref[0] - mean[None, :]) * (inv * g_ref[0, :])[None, :] \
        + bt_ref[0, :][None, :]
    conf = jnp.sum(xtn * cw_ref[0, :][None, :], axis=1) + cb_ref[0, 0]
    wgt = jnp.exp(conf)
    wg = wgt.reshape(HA, 2, WO, 2)
    aw = wg.sum(axis=(1, 3)) + EPS
    nw = wg / aw[:, None, :, None]
    xd = (xtn.reshape(HA, 2, WO, 2, COUT)
          * nw[..., None]).sum(axis=(1, 3)).reshape(HA * WO, COUT)
    xtn_ref[0] = jax.nn.relu(xtn)
    conf_ref[0] = conf[:, None]
    xd_ref[0] = jax.nn.relu(xd)
    nw_ref[0, 0, 0, :] = nw.reshape(TA)


def _apply(xt, stats, gamma, beta, conf_w, conf_b):
    return pl.pallas_call(
        _apply_body,
        grid=(B, N // TA),
        in_specs=[
            pl.BlockSpec((1, TA, COUT), lambda b, t: (b, t, 0)),
            pl.BlockSpec((2, COUT), lambda b, t: (0, 0)),
            pl.BlockSpec((1, COUT), lambda b, t: (0, 0)),
            pl.BlockSpec((1, COUT), lambda b, t: (0, 0)),
            pl.BlockSpec((1, COUT), lambda b, t: (0, 0)),
            pl.BlockSpec((1, 1), lambda b, t: (0, 0)),
        ],
        out_specs=[
            pl.BlockSpec((1, TA, COUT), lambda b, t: (b, t, 0)),
            pl.BlockSpec((1, TA, 1), lambda b, t: (b, t, 0)),
            pl.BlockSpec((1, HA * WO, COUT), lambda b, t: (b, t, 0)),
            pl.BlockSpec((1, 1, 1, TA), lambda b, t: (b, t, 0, 0)),
        ],
        out_shape=[
            jax.ShapeDtypeStruct((B, N, COUT), jnp.float32),
            jax.ShapeDtypeStruct((B, N, 1), jnp.float32),
            jax.ShapeDtypeStruct((B, NS, COUT), jnp.float32),
            jax.ShapeDtypeStruct((B, N // TA, 1, TA), jnp.float32),
        ],
    )(xt, stats, gamma.reshape(1, COUT), beta.reshape(1, COUT),
      conf_w.reshape(1, COUT), conf_b.reshape(1, 1))


def _final_body(idx_ref, nw_ref, aw_ref, awd_ref, iad_ref):
    idx = idx_ref[0, 0, :]
    nw = nw_ref[0, 0, :]
    wt = jnp.zeros((N,), jnp.float32)
    for t in range(N // TN):
        sel = (idx[:, None] ==
               (t * TN + jax.lax.broadcasted_iota(jnp.int32, (N, TN), 1)))
        part = jnp.sum(sel.astype(jnp.float32)
                       * nw[t * TN:(t + 1) * TN][None, :], axis=1)
        wt = wt + part
    awd = aw_ref[0, 0, :] * wt
    awd = awd / jnp.max(awd)
    awd_ref[0, 0, :] = awd
    iad_ref[0, 0, :] = ((idx // W) // 2) * WO + (idx % W) // 2


def _final(idx3, nw3, aw3):
    return pl.pallas_call(
        _final_body,
        grid=(B,),
        in_specs=[
            pl.BlockSpec((1, 1, N), lambda b: (b, 0, 0)),
            pl.BlockSpec((1, 1, N), lambda b: (b, 0, 0)),
            pl.BlockSpec((1, 1, N), lambda b: (b, 0, 0)),
        ],
        out_specs=[
            pl.BlockSpec((1, 1, N), lambda b: (b, 0, 0)),
            pl.BlockSpec((1, 1, N), lambda b: (b, 0, 0)),
        ],
        out_shape=[
            jax.ShapeDtypeStruct((B, 1, N), jnp.float32),
            jax.ShapeDtypeStruct((B, 1, N), jnp.int32),
        ],
    )(idx3, nw3, aw3)


def kernel(x, loc_orig, idx_agg, agg_weight, map_h, map_w,
           conv_w, conv_b, skip_w, gamma, beta, conf_w, conf_b):
    idx_agg = idx_agg.astype(jnp.int32)
    idx56 = _loc_to_idx(loc_orig, H, W)
    idx28 = _loc_to_idx(loc_orig, HO, WO)
    ia3 = idx_agg.reshape(B, 1, N)
    ones3 = jnp.ones((B, 1, N), jnp.float32)
    val3 = agg_weight.reshape(B, 1, N)
    iag_w = idx_agg.reshape(B, NWK, TPW)
    i28_w = idx28.reshape(B, NWK, TPW)

    feat = _sc_gather(x, iag_w, CIN)
    xmap = _scatter_rows(idx56.reshape(B, 1, N), feat, ones3, P, CIN, TN)

    w_taps = conv_w.transpose(2, 3, 1, 0).reshape(9, CIN, COUT)
    ymap = _conv(xmap, w_taps, conv_b)

    feat28 = _sc_gather(ymap, i28_w, COUT)
    xt, stats = _m2t_xt(ia3, feat28, val3, x, skip_w.T)

    xtn, conf, xd, nw4 = _apply(xt, stats, gamma, beta, conf_w, conf_b)

    awd3, iad3 = _final(ia3, nw4.reshape(B, 1, N), val3)

    return (xd, iad3.reshape(B, N), awd3.reshape(B, N, 1),
            conf, xtn)


# speedup vs baseline: 1.1801x; 1.0239x over previous
import functools

import jax
import jax.numpy as jnp
from jax import lax
from jax.experimental import pallas as pl
from jax.experimental.pallas import tpu as pltpu
from jax.experimental.pallas import tpu_sc as plsc

EPS = 1e-6
B, N, CIN, COUT = 4, 3136, 384, 768
H = W = 56
HO = WO = 28
P = H * W
NS = HO * WO
TN = 784
CH = COUT // 2

NWK = 28
TPW = 112


def _loc_to_idx(loc_orig, h, w):
    loc = jnp.clip(loc_orig, -1.0, 1.0)
    xs = 0.5 * (loc[..., 0] + 1.0) * w - 0.5
    ys = 0.5 * (loc[..., 1] + 1.0) * h - 0.5
    xg = jnp.clip(jnp.round(xs).astype(jnp.int32), 0, w - 1)
    yg = jnp.clip(jnp.round(ys).astype(jnp.int32), 0, h - 1)
    return yg * w + xg


_SC_MESH = plsc.VectorSubcoreMesh(core_axis_name="c", subcore_axis_name="s")


def _sc_gather_body(tab_hbm, idx_hbm, out_hbm, rows0, rows1, idx_v,
                    sem0, sem1, *, c, cr):
    cc = lax.axis_index("c")
    s = lax.axis_index("s")
    w = s * 2 + cc
    bufs, sems = (rows0, rows1), (sem0, sem1)
    units = [(b, k) for b in range(B) for k in range(TPW // cr)]

    @pl.when(w < NWK)
    def _():
        pltpu.sync_copy(idx_hbm.at[w], idx_v)

        def fire(u, i):
            b, k = units[i]
            return pltpu.async_copy(
                tab_hbm.at[b].at[idx_v.at[pl.ds(b * TPW + k * cr, cr)]],
                bufs[u], sems[u])

        def drain(u, i):
            b, k = units[i]
            pltpu.sync_copy(bufs[u],
                            out_hbm.at[b, pl.ds(w * TPW + k * cr, cr)])

        cp = fire(0, 0)
        for i in range(1, len(units)):
            nxt = fire(i % 2, i)
            cp.wait()
            drain((i - 1) % 2, i - 1)
            cp = nxt
        cp.wait()
        drain((len(units) - 1) % 2, len(units) - 1)


def _sc_gather(tab, idx_w, c):
    cr = TPW if c <= 384 else TPW // 2
    f = pl.kernel(
        functools.partial(_sc_gather_body, c=c, cr=cr),
        out_type=jax.ShapeDtypeStruct((B, N, c), jnp.float32),
        mesh=_SC_MESH,
        scratch_types=[
            pltpu.VMEM((cr, c), jnp.float32),
            pltpu.VMEM((cr, c), jnp.float32),
            pltpu.VMEM((B * TPW,), jnp.int32),
            pltpu.SemaphoreType.DMA,
            pltpu.SemaphoreType.DMA,
        ],
    )
    return f(tab, idx_w)


def _scatter_rows_body(idx_ref, z_ref, val_ref, o_ref, *, tp):
    pt = pl.program_id(1)
    idx = idx_ref[0, 0, :]
    sel = (idx[:, None] ==
           (pt * tp + jax.lax.broadcasted_iota(jnp.int32, (N, tp), 1)))
    val = val_ref[0, 0, :]
    sv = sel.astype(jnp.bfloat16) * val.astype(jnp.bfloat16)[:, None]
    acc = jax.lax.dot_general(sv, z_ref[0].astype(jnp.bfloat16),
                              (((0,), (0,)), ((), ())),
                              preferred_element_type=jnp.float32)
    ones = jnp.ones((N, 128), jnp.bfloat16)
    tot = jax.lax.dot_general(sv, ones, (((0,), (0,)), ((), ())),
                              preferred_element_type=jnp.float32)[:, :1]
    o_ref[0] = acc / (tot + EPS)


def _scatter_rows(idx3, z, val3, nbins, c, tp):
    return pl.pallas_call(
        functools.partial(_scatter_rows_body, tp=tp),
        grid=(B, nbins // tp),
        in_specs=[
            pl.BlockSpec((1, 1, N), lambda b, t: (b, 0, 0)),
            pl.BlockSpec((1, N, c), lambda b, t: (b, 0, 0)),
            pl.BlockSpec((1, 1, N), lambda b, t: (b, 0, 0)),
        ],
        out_specs=pl.BlockSpec((1, tp, c), lambda b, t: (b, t, 0)),
        out_shape=jax.ShapeDtypeStruct((B, nbins, c), jnp.float32),
    )(idx3, z, val3)


def _conv_body(xm_ref, w_ref, b_ref, o_ref):
    xr = xm_ref[0].reshape(HO, 2, WO, 2, CIN)
    out = jnp.broadcast_to(b_ref[0, :], (NS, CH))

    def shifted(py, px, sy, sx):
        a = xr[:, py, :, px, :]
        if sy:
            a = jnp.concatenate([jnp.zeros((1, WO, CIN), jnp.float32),
                                 a[:-1]], axis=0)
        if sx:
            a = jnp.concatenate([jnp.zeros((HO, 1, CIN), jnp.float32),
                                 a[:, :-1]], axis=1)
        return a.reshape(NS, CIN)

    for ky, (py, sy) in enumerate(((1, True), (0, False), (1, False))):
        for kx, (px, sx) in enumerate(((1, True), (0, False), (1, False))):
            a = shifted(py, px, sy, sx)
            out = out + jnp.dot(a, w_ref[ky * 3 + kx],
                                preferred_element_type=jnp.float32)
    o_ref[0] = out


def _conv(xmap, w_taps, conv_b):
    return pl.pallas_call(
        _conv_body,
        grid=(B, COUT // CH),
        in_specs=[
            pl.BlockSpec((1, P, CIN), lambda b, h: (b, 0, 0)),
            pl.BlockSpec((9, CIN, CH), lambda b, h: (0, 0, h)),
            pl.BlockSpec((1, CH), lambda b, h: (0, h)),
        ],
        out_specs=pl.BlockSpec((1, NS, CH), lambda b, h: (b, 0, h)),
        out_shape=jax.ShapeDtypeStruct((B, NS, COUT), jnp.float32),
    )(xmap, w_taps, conv_b.reshape(1, COUT))


def _m2t_xt_body(idx_ref, z_ref, val_ref, x_ref, sw_ref, xt_ref, s_ref):
    pt = pl.program_id(1)
    idx = idx_ref[0, 0, :]
    sel = (idx[:, None] ==
           (pt * TN + jax.lax.broadcasted_iota(jnp.int32, (N, TN), 1)))
    val = val_ref[0, 0, :]
    sv = sel.astype(jnp.bfloat16) * val.astype(jnp.bfloat16)[:, None]
    acc = jax.lax.dot_general(sv, z_ref[0].astype(jnp.bfloat16),
                              (((0,), (0,)), ((), ())),
                              preferred_element_type=jnp.float32)
    ones = jnp.ones((N, 128), jnp.bfloat16)
    tot = jax.lax.dot_general(sv, ones, (((0,), (0,)), ((), ())),
                              preferred_element_type=jnp.float32)[:, :1]
    xt = (acc / (tot + EPS)
          + jnp.dot(x_ref[0], sw_ref[...],
                    preferred_element_type=jnp.float32))
    xt_ref[0] = xt
    s0 = jnp.sum(xt, axis=0)
    s1 = jnp.sum(xt * xt, axis=0)
    st = jnp.stack([s0, s1], axis=0)
    first = (pl.program_id(0) == 0) & (pl.program_id(1) == 0)

    @pl.when(first)
    def _():
        s_ref[...] = st

    @pl.when(jnp.logical_not(first))
    def _():
        s_ref[...] += st


def _m2t_xt(idx3, z, val3, x, skip_wt):
    return pl.pallas_call(
        _m2t_xt_body,
        grid=(B, N // TN),
        in_specs=[
            pl.BlockSpec((1, 1, N), lambda b, t: (b, 0, 0)),
            pl.BlockSpec((1, N, COUT), lambda b, t: (b, 0, 0)),
            pl.BlockSpec((1, 1, N), lambda b, t: (b, 0, 0)),
            pl.BlockSpec((1, TN, CIN), lambda b, t: (b, t, 0)),
            pl.BlockSpec((CIN, COUT), lambda b, t: (0, 0)),
        ],
        out_specs=[
            pl.BlockSpec((1, TN, COUT), lambda b, t: (b, t, 0)),
            pl.BlockSpec((2, COUT), lambda b, t: (0, 0)),
        ],
        out_shape=[
            jax.ShapeDtypeStruct((B, N, COUT), jnp.float32),
            jax.ShapeDtypeStruct((2, COUT), jnp.float32),
        ],
    )(idx3, z, val3, x, skip_wt)


TA = N // 2
HA = HO // 2


def _apply_body(xt_ref, s_ref, g_ref, bt_ref, cw_ref, cb_ref,
                xtn_ref, conf_ref, xd_ref, nw_ref):
    cnt = B * N
    mean = s_ref[0, :] / cnt
    var = s_ref[1, :] / cnt - mean * mean
    inv = jax.lax.rsqrt(var + 1e-5)
    xtn = (xt_ref[0] - mean[None, :]) * (inv * g_ref[0, :])[None, :] \
        + bt_ref[0, :][None, :]
    conf = jnp.sum(xtn * cw_ref[0, :][None, :], axis=1) + cb_ref[0, 0]
    wgt = jnp.exp(conf)
    wg = wgt.reshape(HA, 2, WO, 2)
    aw = wg.sum(axis=(1, 3)) + EPS
    nw = wg / aw[:, None, :, None]
    xd = (xtn.reshape(HA, 2, WO, 2, COUT)
          * nw[..., None]).sum(axis=(1, 3)).reshape(HA * WO, COUT)
    xtn_ref[0] = jax.nn.relu(xtn)
    conf_ref[0] = conf[:, None]
    xd_ref[0] = jax.nn.relu(xd)
    nw_ref[0, 0, 0, :] = nw.reshape(TA)


def _apply(xt, stats, gamma, beta, conf_w, conf_b):
    return pl.pallas_call(
        _apply_body,
        grid=(B, N // TA),
        in_specs=[
            pl.BlockSpec((1, TA, COUT), lambda b, t: (b, t, 0)),
            pl.BlockSpec((2, COUT), lambda b, t: (0, 0)),
            pl.BlockSpec((1, COUT), lambda b, t: (0, 0)),
            pl.BlockSpec((1, COUT), lambda b, t: (0, 0)),
            pl.BlockSpec((1, COUT), lambda b, t: (0, 0)),
            pl.BlockSpec((1, 1), lambda b, t: (0, 0)),
        ],
        out_specs=[
            pl.BlockSpec((1, TA, COUT), lambda b, t: (b, t, 0)),
            pl.BlockSpec((1, TA, 1), lambda b, t: (b, t, 0)),
            pl.BlockSpec((1, HA * WO, COUT), lambda b, t: (b, t, 0)),
            pl.BlockSpec((1, 1, 1, TA), lambda b, t: (b, t, 0, 0)),
        ],
        out_shape=[
            jax.ShapeDtypeStruct((B, N, COUT), jnp.float32),
            jax.ShapeDtypeStruct((B, N, 1), jnp.float32),
            jax.ShapeDtypeStruct((B, NS, COUT), jnp.float32),
            jax.ShapeDtypeStruct((B, N // TA, 1, TA), jnp.float32),
        ],
    )(xt, stats, gamma.reshape(1, COUT), beta.reshape(1, COUT),
      conf_w.reshape(1, COUT), conf_b.reshape(1, 1))


def _final_body(idx_ref, nw_ref, aw_ref, awd_ref, iad_ref):
    idx = idx_ref[0, 0, :]
    nw = nw_ref[0, 0, :]
    wt = jnp.zeros((N,), jnp.float32)
    for t in range(N // TN):
        sel = (idx[:, None] ==
               (t * TN + jax.lax.broadcasted_iota(jnp.int32, (N, TN), 1)))
        part = jnp.sum(sel.astype(jnp.float32)
                       * nw[t * TN:(t + 1) * TN][None, :], axis=1)
        wt = wt + part
    awd = aw_ref[0, 0, :] * wt
    awd = awd / jnp.max(awd)
    awd_ref[0, 0, :] = awd
    iad_ref[0, 0, :] = ((idx // W) // 2) * WO + (idx % W) // 2


def _final(idx3, nw3, aw3):
    return pl.pallas_call(
        _final_body,
        grid=(B,),
        in_specs=[
            pl.BlockSpec((1, 1, N), lambda b: (b, 0, 0)),
            pl.BlockSpec((1, 1, N), lambda b: (b, 0, 0)),
            pl.BlockSpec((1, 1, N), lambda b: (b, 0, 0)),
        ],
        out_specs=[
            pl.BlockSpec((1, 1, N), lambda b: (b, 0, 0)),
            pl.BlockSpec((1, 1, N), lambda b: (b, 0, 0)),
        ],
        out_shape=[
            jax.ShapeDtypeStruct((B, 1, N), jnp.float32),
            jax.ShapeDtypeStruct((B, 1, N), jnp.int32),
        ],
    )(idx3, nw3, aw3)


def kernel(x, loc_orig, idx_agg, agg_weight, map_h, map_w,
           conv_w, conv_b, skip_w, gamma, beta, conf_w, conf_b):
    idx_agg = idx_agg.astype(jnp.int32)
    idx56 = _loc_to_idx(loc_orig, H, W)
    idx28 = _loc_to_idx(loc_orig, HO, WO)
    ia3 = idx_agg.reshape(B, 1, N)
    ones3 = jnp.ones((B, 1, N), jnp.float32)
    val3 = agg_weight.reshape(B, 1, N)
    iag_w = idx_agg.reshape(B, NWK, TPW).transpose(1, 0, 2) \
        .reshape(NWK, B * TPW)
    i28_w = idx28.reshape(B, NWK, TPW).transpose(1, 0, 2) \
        .reshape(NWK, B * TPW)

    feat = _sc_gather(x, iag_w, CIN)
    xmap = _scatter_rows(idx56.reshape(B, 1, N), feat, ones3, P, CIN, TN)

    w_taps = conv_w.transpose(2, 3, 1, 0).reshape(9, CIN, COUT)
    ymap = _conv(xmap, w_taps, conv_b)

    feat28 = _sc_gather(ymap, i28_w, COUT)
    xt, stats = _m2t_xt(ia3, feat28, val3, x, skip_w.T)

    xtn, conf, xd, nw4 = _apply(xt, stats, gamma, beta, conf_w, conf_b)

    awd3, iad3 = _final(ia3, nw4.reshape(B, 1, N), val3)

    return (xd, iad3.reshape(B, N), awd3.reshape(B, N, 1),
            conf, xtn)
